# Initial kernel scaffold; baseline (speedup 1.0000x reference)
import jax
import jax.numpy as jnp
from jax.experimental import pallas as pl


def kernel(keys, values, query):
    raise NotImplementedError("write your pallas kernel here")


# speedup vs baseline: 1.3889x; 1.3889x over previous
import functools

import jax
import jax.numpy as jnp
import numpy as np
from jax.experimental import pallas as pl
from jax.experimental.pallas import tpu as pltpu


def _importance_select_body(q_ref, k_ref, w_ref, evict_ref, logits_scr,
                            *, n_steps, k_heavy, n_recent, scale):
    i = pl.program_id(0)
    l_blk = jax.lax.dot_general(
        q_ref[...], k_ref[...], (((1,), (1,)), ((), ())),
        preferred_element_type=jnp.float32) * scale
    bl = l_blk.shape[1]
    logits_scr[:, pl.ds(i * bl, bl)] = l_blk

    @pl.when(i == n_steps - 1)
    def _():
        logits = logits_scr[...]
        m = jnp.max(logits, axis=1, keepdims=True)
        e = jnp.exp(logits - m)
        s = jnp.sum(e, axis=1, keepdims=True)
        imp = jnp.sum(e / s, axis=0, keepdims=True)
        L = imp.shape[1]

        u = jax.lax.bitcast_convert_type(imp, jnp.int32)

        def t_step(j, t):
            cand = t | (jnp.int32(1) << (30 - j))
            cnt = jnp.sum((u >= cand).astype(jnp.int32))
            return jnp.where(cnt >= k_heavy, cand, t)
        T = jax.lax.fori_loop(0, 31, t_step, jnp.int32(0))

        eq = u == T
        c_gt = jnp.sum((u > T).astype(jnp.int32))
        need_eq = k_heavy - c_gt
        idx = jax.lax.broadcasted_iota(jnp.int32, (1, L), 1)

        def j_step(j, lohi):
            lo, hi = lohi
            mid = (lo + hi) // 2
            cnt = jnp.sum((eq & (idx <= mid)).astype(jnp.int32))
            pred = cnt >= need_eq
            return (jnp.where(pred, lo, mid + 1), jnp.where(pred, mid, hi))
        J, _ = jax.lax.fori_loop(0, 13, j_step,
                                 (jnp.int32(0), jnp.int32(L - 1)))

        keep = (u > T) | (eq & (idx <= J)) | (idx >= L - n_recent)
        w_ref[...] = imp * keep.astype(jnp.float32)
        evict_ref[...] = jnp.logical_not(keep).astype(jnp.int32)


def _scale_body(v_ref, w_ref, o_ref):
    o_ref[...] = v_ref[...] * w_ref[...]


def kernel(keys, values, query):
    L, H = keys.shape
    B = query.shape[0]
    k_heavy = max(1, int(L * 0.5))
    n_recent = max(1, int(L * 0.25))
    scale = 1.0 / np.sqrt(H)

    BL = 1024
    n_steps = L // BL
    w, evict = pl.pallas_call(
        functools.partial(_importance_select_body, n_steps=n_steps,
                          k_heavy=k_heavy, n_recent=n_recent, scale=scale),
        grid=(n_steps,),
        in_specs=[pl.BlockSpec((B, H), lambda i: (0, 0)),
                  pl.BlockSpec((BL, H), lambda i: (i, 0))],
        out_specs=[pl.BlockSpec((1, L), lambda i: (0, 0)),
                   pl.BlockSpec((1, L), lambda i: (0, 0))],
        out_shape=[jax.ShapeDtypeStruct((1, L), jnp.float32),
                   jax.ShapeDtypeStruct((1, L), jnp.int32)],
        scratch_shapes=[pltpu.VMEM((B, L), jnp.float32)],
    )(query, keys)

    RB = 1024
    weighted = pl.pallas_call(
        _scale_body,
        grid=(L // RB,),
        in_specs=[pl.BlockSpec((RB, H), lambda i: (i, 0)),
                  pl.BlockSpec((RB, 1), lambda i: (i, 0))],
        out_specs=pl.BlockSpec((RB, H), lambda i: (i, 0)),
        out_shape=jax.ShapeDtypeStruct((L, H), jnp.float32),
    )(values, w.reshape(L, 1))

    evict_mask = evict.reshape(L) != 0
    return evict_mask, weighted


# speedup vs baseline: 1.4832x; 1.0679x over previous
import functools

import jax
import jax.numpy as jnp
import numpy as np
from jax.experimental import pallas as pl
from jax.experimental.pallas import tpu as pltpu


def _fused_body(q_ref, k_ref, v_ref, evict_ref, o_ref, logits_scr, w_scr,
                *, n_blk, blk, k_heavy, n_recent, scale):
    i = pl.program_id(0)

    @pl.when(i < n_blk)
    def _matmul_step():
        l_blk = jax.lax.dot_general(
            q_ref[...], k_ref[...], (((1,), (1,)), ((), ())),
            preferred_element_type=jnp.float32) * scale
        logits_scr[:, pl.ds(i * blk, blk)] = l_blk

    @pl.when(i == n_blk - 1)
    def _select_step():
        logits = logits_scr[...]
        m = jnp.max(logits, axis=1, keepdims=True)
        e = jnp.exp(logits - m)
        s = jnp.sum(e, axis=1, keepdims=True)
        imp = jnp.sum(e / s, axis=0, keepdims=True)
        L = imp.shape[1]

        u = jax.lax.bitcast_convert_type(imp, jnp.int32)

        def t_step(j, t):
            cand = t | (jnp.int32(1) << (30 - j))
            cnt = jnp.sum((u >= cand).astype(jnp.int32))
            return jnp.where(cnt >= k_heavy, cand, t)
        T = jax.lax.fori_loop(0, 31, t_step, jnp.int32(0))

        eq = u == T
        c_gt = jnp.sum((u > T).astype(jnp.int32))
        c_eq = jnp.sum(eq.astype(jnp.int32))
        need_eq = k_heavy - c_gt
        idx = jax.lax.broadcasted_iota(jnp.int32, (1, L), 1)

        def j_search(_):
            def j_step(j, lohi):
                lo, hi = lohi
                mid = (lo + hi) // 2
                cnt = jnp.sum((eq & (idx <= mid)).astype(jnp.int32))
                pred = cnt >= need_eq
                return (jnp.where(pred, lo, mid + 1), jnp.where(pred, mid, hi))
            lo, _ = jax.lax.fori_loop(0, 13, j_step,
                                      (jnp.int32(0), jnp.int32(L - 1)))
            return lo
        J = jax.lax.cond(c_eq > need_eq, j_search,
                         lambda _: jnp.int32(L - 1), 0)

        keep = (u > T) | (eq & (idx <= J)) | (idx >= L - n_recent)
        w_scr[...] = imp * keep.astype(jnp.float32)
        evict_ref[...] = jnp.logical_not(keep).astype(jnp.int32)

    @pl.when(i >= n_blk)
    def _scale_step():
        j = i - n_blk
        w_col = jnp.transpose(w_scr[:, pl.ds(j * blk, blk)], (1, 0))
        o_ref[...] = v_ref[...] * w_col


def kernel(keys, values, query):
    L, H = keys.shape
    B = query.shape[0]
    k_heavy = max(1, int(L * 0.5))
    n_recent = max(1, int(L * 0.25))
    scale = 1.0 / np.sqrt(H)

    BLK = 1024
    n_blk = L // BLK
    evict, weighted = pl.pallas_call(
        functools.partial(_fused_body, n_blk=n_blk, blk=BLK, k_heavy=k_heavy,
                          n_recent=n_recent, scale=scale),
        grid=(2 * n_blk,),
        in_specs=[
            pl.BlockSpec((B, H), lambda i: (0, 0)),
            pl.BlockSpec((BLK, H), lambda i: (jnp.minimum(i, n_blk - 1), 0)),
            pl.BlockSpec((BLK, H), lambda i: (jnp.maximum(i - n_blk, 0), 0)),
        ],
        out_specs=[
            pl.BlockSpec((1, L), lambda i: (0, 0)),
            pl.BlockSpec((BLK, H), lambda i: (jnp.maximum(i - n_blk, 0), 0)),
        ],
        out_shape=[
            jax.ShapeDtypeStruct((1, L), jnp.int32),
            jax.ShapeDtypeStruct((L, H), jnp.float32),
        ],
        scratch_shapes=[pltpu.VMEM((B, L), jnp.float32),
                        pltpu.VMEM((1, L), jnp.float32)],
    )(query, keys, values)

    evict_mask = evict.reshape(L) != 0
    return evict_mask, weighted
